# Initial kernel scaffold; baseline (speedup 1.0000x reference)
import functools

import jax
import jax.numpy as jnp
from jax import lax
from jax.experimental import pallas as pl
from jax.experimental.pallas import tpu as pltpu
from jax.experimental.pallas import tpu_sc as plsc

NCORE = 2
NSUB = 16
EB = 128


def _make_deg(n, npad, epad):
    ept = epad // (NCORE * NSUB)
    nb = ept // EB
    zrows = npad // NSUB
    mesh = plsc.VectorSubcoreMesh(core_axis_name="c", subcore_axis_name="s")

    @functools.partial(
        pl.kernel,
        out_type=jax.ShapeDtypeStruct((NCORE * npad, 16), jnp.float32),
        mesh=mesh,
        scratch_types=[
            pltpu.VMEM_SHARED((npad, 16), jnp.float32),
            pltpu.VMEM((zrows, 16), jnp.float32),
            pltpu.VMEM((EB, 16), jnp.float32),
            pltpu.VMEM((EB,), jnp.int32),
        ],
    )
    def deg(dst_hbm, deg_hbm, dacc, zbuf, ones, idx):
        c = lax.axis_index("c")
        s = lax.axis_index("s")

        def fill(i, _):
            zbuf[i] = jnp.zeros((16,), jnp.float32)
            return 0
        lax.fori_loop(0, zrows, fill, 0)

        def fill1(i, _):
            ones[i] = jnp.ones((16,), jnp.float32)
            return 0
        lax.fori_loop(0, EB, fill1, 0)

        pltpu.sync_copy(zbuf, dacc.at[pl.ds(s * zrows, zrows)])
        plsc.subcore_barrier()

        ebase = c * (epad // NCORE) + s * ept

        def bb(b, _):
            pltpu.sync_copy(dst_hbm.at[pl.ds(ebase + b * EB, EB)], idx)
            pltpu.sync_copy(ones, dacc.at[idx], add=True)
            return 0
        lax.fori_loop(0, nb, bb, 0)
        plsc.subcore_barrier()

        pltpu.sync_copy(dacc.at[pl.ds(s * zrows, zrows)],
                        deg_hbm.at[pl.ds(c * npad + s * zrows, zrows)])

    return deg


def _make_scatter(nch, n, npad, epad):
    per_core = nch // NCORE
    ept = epad // NSUB
    nb = ept // EB
    zrows = npad // NSUB
    zh = zrows // 2
    frows = n // NSUB
    mesh = plsc.VectorSubcoreMesh(core_axis_name="c", subcore_axis_name="s")

    @functools.partial(
        pl.kernel,
        out_type=jax.ShapeDtypeStruct((nch, n, 128), jnp.float32),
        mesh=mesh,
        scratch_types=[
            pltpu.VMEM_SHARED((npad, 128), jnp.float32),
            pltpu.VMEM((zh, 128), jnp.float32),
            pltpu.VMEM((EB, 128), jnp.float32),
            pltpu.VMEM((EB,), jnp.int32),
            pltpu.VMEM((EB,), jnp.int32),
            pltpu.SemaphoreType.DMA,
        ],
    )
    def scatter(g_hbm, src_hbm, dst_hbm, p_hbm, acc, zbuf, rows, isrc, idst,
                sem):
        c = lax.axis_index("c")
        s = lax.axis_index("s")

        def fill(i, _):
            for k8 in range(8):
                zbuf[i, pl.ds(k8 * 16, 16)] = jnp.zeros((16,), jnp.float32)
            return 0
        lax.fori_loop(0, zh, fill, 0)

        ebase = s * ept

        def process(k):
            pltpu.sync_copy(zbuf, acc.at[pl.ds(s * zrows, zh)])
            pltpu.sync_copy(zbuf, acc.at[pl.ds(s * zrows + zh, zh)])
            plsc.subcore_barrier()

            def bb(b, _):
                base = ebase + b * EB
                pltpu.sync_copy(src_hbm.at[pl.ds(base, EB)], isrc)
                pltpu.sync_copy(dst_hbm.at[pl.ds(base, EB)], idst)
                pltpu.async_copy(g_hbm.at[k].at[isrc], rows, sem).wait()
                pltpu.sync_copy(rows, acc.at[idst], add=True)
                return 0
            lax.fori_loop(0, nb, bb, 0)
            plsc.subcore_barrier()

            pltpu.sync_copy(acc.at[pl.ds(s * frows, frows)],
                            p_hbm.at[k].at[pl.ds(s * frows, frows)])
            plsc.subcore_barrier()

        for ci in range(per_core):
            @pl.when(c == 0)
            def _():
                process(ci)

            @pl.when(c == 1)
            def _():
                process(per_core + ci)

    return scatter


def _dinv_call(deg16, n, npad):
    def body(deg_ref, out_ref):
        cnt = deg_ref[0:n, 0] + deg_ref[npad:npad + n, 0] + 1.0
        out_ref[:, 0] = lax.rsqrt(cnt)

    return pl.pallas_call(
        body,
        out_shape=jax.ShapeDtypeStruct((n, 1), jnp.float32),
    )(deg16)


def _mm0_call(x, dinv, W, R, bn):
    n, d_in = x.shape
    dh = W.shape[1]
    nch = dh // 128
    grid = n // bn

    def body(x_ref, dv_ref, w_ref, r_ref, g_ref, res_ref):
        xb = x_ref[...]
        g = jnp.dot(xb, w_ref[...], preferred_element_type=jnp.float32)
        g = g * dv_ref[...]
        for k in range(nch):
            g_ref[k] = g[:, k * 128:(k + 1) * 128]
        res_ref[...] = jnp.dot(xb, r_ref[...],
                               preferred_element_type=jnp.float32)

    return pl.pallas_call(
        body,
        grid=(grid,),
        in_specs=[
            pl.BlockSpec((bn, d_in), lambda i: (i, 0)),
            pl.BlockSpec((bn, 1), lambda i: (i, 0)),
            pl.BlockSpec((d_in, dh), lambda i: (0, 0)),
            pl.BlockSpec((d_in, dh), lambda i: (0, 0)),
        ],
        out_specs=[
            pl.BlockSpec((nch, bn, 128), lambda i: (0, i, 0)),
            pl.BlockSpec((bn, dh), lambda i: (i, 0)),
        ],
        out_shape=[
            jax.ShapeDtypeStruct((nch, n, 128), jnp.float32),
            jax.ShapeDtypeStruct((n, dh), jnp.float32),
        ],
    )(x, dinv, W, R)


def _comb_mm_call(p_ch, g_ch, res, dinv, b, rb, W, R, bn):
    nch_in, n, _ = p_ch.shape
    d_in = nch_in * 128
    dh = W.shape[1]
    nch_out = dh // 128
    grid = n // bn

    def body(p_ref, g_ref, res_ref, dv_ref, b_ref, rb_ref, w_ref, r_ref,
             go_ref, ro_ref):
        h = jnp.concatenate(
            [p_ref[k] + g_ref[k] for k in range(nch_in)], axis=1)
        h = h * dv_ref[...] + b_ref[...] + res_ref[...] + rb_ref[...]
        h = jnp.maximum(h, 0.0)
        g2 = jnp.dot(h, w_ref[...], preferred_element_type=jnp.float32)
        g2 = g2 * dv_ref[...]
        for k in range(nch_out):
            go_ref[k] = g2[:, k * 128:(k + 1) * 128]
        ro_ref[...] = jnp.dot(h, r_ref[...],
                              preferred_element_type=jnp.float32)

    return pl.pallas_call(
        body,
        grid=(grid,),
        in_specs=[
            pl.BlockSpec((nch_in, bn, 128), lambda i: (0, i, 0)),
            pl.BlockSpec((nch_in, bn, 128), lambda i: (0, i, 0)),
            pl.BlockSpec((bn, d_in), lambda i: (i, 0)),
            pl.BlockSpec((bn, 1), lambda i: (i, 0)),
            pl.BlockSpec((d_in,), lambda i: (0,)),
            pl.BlockSpec((d_in,), lambda i: (0,)),
            pl.BlockSpec((d_in, dh), lambda i: (0, 0)),
            pl.BlockSpec((d_in, dh), lambda i: (0, 0)),
        ],
        out_specs=[
            pl.BlockSpec((nch_out, bn, 128), lambda i: (0, i, 0)),
            pl.BlockSpec((bn, dh), lambda i: (i, 0)),
        ],
        out_shape=[
            jax.ShapeDtypeStruct((nch_out, n, 128), jnp.float32),
            jax.ShapeDtypeStruct((n, dh), jnp.float32),
        ],
    )(p_ch, g_ch, res, dinv, b, rb, W, R)


def _final_call(p_ch, g_ch, res, dinv, b, rb, bn):
    nch_in, n, _ = p_ch.shape
    d_in = nch_in * 128
    grid = n // bn

    def body(p_ref, g_ref, res_ref, dv_ref, b_ref, rb_ref, out_ref):
        h = jnp.concatenate(
            [p_ref[k] + g_ref[k] for k in range(nch_in)], axis=1)
        out_ref[...] = (h * dv_ref[...] + b_ref[...] + res_ref[...]
                        + rb_ref[...])

    return pl.pallas_call(
        body,
        grid=(grid,),
        in_specs=[
            pl.BlockSpec((nch_in, bn, 128), lambda i: (0, i, 0)),
            pl.BlockSpec((nch_in, bn, 128), lambda i: (0, i, 0)),
            pl.BlockSpec((bn, d_in), lambda i: (i, 0)),
            pl.BlockSpec((bn, 1), lambda i: (i, 0)),
            pl.BlockSpec((d_in,), lambda i: (0,)),
            pl.BlockSpec((d_in,), lambda i: (0,)),
        ],
        out_specs=pl.BlockSpec((bn, d_in), lambda i: (i, 0)),
        out_shape=jax.ShapeDtypeStruct((n, d_in), jnp.float32),
    )(p_ch, g_ch, res, dinv, b, rb)


def kernel(x, edge_index, W0, b0, W1, b1, W2, b2, R0, rb0, R1, rb1, R2, rb2):
    n = x.shape[0]
    e = edge_index.shape[1]
    npad = n + NSUB
    gran = NCORE * NSUB * EB
    epad = ((e + gran - 1) // gran) * gran
    bn = 1000 if n % 1000 == 0 else n // 8

    src = jnp.pad(edge_index[0], (0, epad - e))
    dst = jnp.pad(edge_index[1], (0, epad - e), constant_values=n)

    deg16 = _make_deg(n, npad, epad)(dst)
    dinv = _dinv_call(deg16, n, npad)

    g0, res0 = _mm0_call(x, dinv, W0, R0, bn)
    p0 = _make_scatter(g0.shape[0], n, npad, epad)(g0, src, dst)
    g1, res1 = _comb_mm_call(p0, g0, res0, dinv, b0, rb0, W1, R1, bn)
    p1 = _make_scatter(g1.shape[0], n, npad, epad)(g1, src, dst)
    g2, res2 = _comb_mm_call(p1, g1, res1, dinv, b1, rb1, W2, R2, bn)
    p2 = _make_scatter(g2.shape[0], n, npad, epad)(g2, src, dst)
    out = _final_call(p2, g2, res2, dinv, b2, rb2, bn)
    return out


# speedup vs baseline: 5.5323x; 5.5323x over previous
import functools

import jax
import jax.numpy as jnp
from jax import lax
from jax.experimental import pallas as pl
from jax.experimental.pallas import tpu as pltpu
from jax.experimental.pallas import tpu_sc as plsc

NCORE = 2
NSUB = 16
EB = 128
CW = 64


def _make_deg(n, npad, epad):
    ept = epad // (NCORE * NSUB)
    nb = ept // EB
    zrows = npad // NSUB
    mesh = plsc.VectorSubcoreMesh(core_axis_name="c", subcore_axis_name="s",
                                  num_cores=NCORE, num_subcores=NSUB)

    @functools.partial(
        pl.kernel,
        out_type=jax.ShapeDtypeStruct((NCORE * npad, 16), jnp.float32),
        mesh=mesh,
        scratch_types=[
            pltpu.VMEM_SHARED((npad, 16), jnp.float32),
            pltpu.VMEM((zrows, 16), jnp.float32),
            pltpu.VMEM((EB, 16), jnp.float32),
            pltpu.VMEM((EB,), jnp.int32),
        ],
        compiler_params=pltpu.CompilerParams(use_tc_tiling_on_sc=False),
    )
    def deg(dst_hbm, deg_hbm, dacc, zbuf, ones, idx):
        c = lax.axis_index("c")
        s = lax.axis_index("s")

        def fill(i, _):
            zbuf[i] = jnp.zeros((16,), jnp.float32)
            return 0
        lax.fori_loop(0, zrows, fill, 0)

        def fill1(i, _):
            ones[i] = jnp.ones((16,), jnp.float32)
            return 0
        lax.fori_loop(0, EB, fill1, 0)

        pltpu.sync_copy(zbuf, dacc.at[pl.ds(s * zrows, zrows)])
        plsc.subcore_barrier()

        ebase = c * (epad // NCORE) + s * ept

        def bb(b, _):
            pltpu.sync_copy(dst_hbm.at[pl.ds(ebase + b * EB, EB)], idx)
            pltpu.sync_copy(ones, dacc.at[idx], add=True)
            return 0
        lax.fori_loop(0, nb, bb, 0)
        plsc.subcore_barrier()

        pltpu.sync_copy(dacc.at[pl.ds(s * zrows, zrows)],
                        deg_hbm.at[pl.ds(c * npad + s * zrows, zrows)])

    return deg


def _make_scatter(nch, n, npad, epad):
    per_core = nch // NCORE
    ept = epad // NSUB
    nb = ept // EB
    zrows = npad // NSUB
    nbuf = 2
    assert nb % nbuf == 0
    mesh = plsc.VectorSubcoreMesh(core_axis_name="c", subcore_axis_name="s",
                                  num_cores=NCORE, num_subcores=NSUB)

    @functools.partial(
        pl.kernel,
        out_type=jax.ShapeDtypeStruct((nch, npad, CW), jnp.float32),
        mesh=mesh,
        scratch_types=[
            pltpu.VMEM_SHARED((npad, CW), jnp.float32),
            pltpu.VMEM((nbuf, EB, CW), jnp.float32),
            pltpu.VMEM((nb, EB), jnp.int32),
            pltpu.VMEM((nb, EB), jnp.int32),
            pltpu.SemaphoreType.DMA((nbuf,)),
        ],
        compiler_params=pltpu.CompilerParams(use_tc_tiling_on_sc=False),
    )
    def scatter(g_hbm, src_hbm, dst_hbm, z_hbm, p_hbm, acc, rows, isrc, idst,
                gsem):
        c = lax.axis_index("c")
        s = lax.axis_index("s")

        pltpu.sync_copy(src_hbm.at[pl.ds(s * nb, nb)], isrc)
        pltpu.sync_copy(dst_hbm.at[pl.ds(s * nb, nb)], idst)

        tl = NSUB - 1
        grows = n - tl * zrows

        def process(k):
            @pl.when(s < tl)
            def _():
                pltpu.sync_copy(g_hbm.at[k].at[pl.ds(s * zrows, zrows)],
                                acc.at[pl.ds(s * zrows, zrows)])

            @pl.when(s == tl)
            def _():
                pltpu.sync_copy(g_hbm.at[k].at[pl.ds(tl * zrows, grows)],
                                acc.at[pl.ds(tl * zrows, grows)])
                pltpu.sync_copy(z_hbm, acc.at[pl.ds(n, npad - n)])
            plsc.subcore_barrier()

            for j in range(nbuf):
                pltpu.async_copy(g_hbm.at[k].at[isrc.at[j]], rows.at[j],
                                 gsem.at[j])

            def group(g, _):
                for j in range(nbuf):
                    b = g * nbuf + j
                    pltpu.make_async_copy(g_hbm.at[k].at[isrc.at[b]],
                                          rows.at[j], gsem.at[j]).wait()
                    pltpu.sync_copy(rows.at[j], acc.at[idst.at[b]], add=True)

                    @pl.when(b + nbuf < nb)
                    def _():
                        pltpu.async_copy(g_hbm.at[k].at[isrc.at[b + nbuf]],
                                         rows.at[j], gsem.at[j])
                return 0
            lax.fori_loop(0, nb // nbuf, group, 0)
            plsc.subcore_barrier()

            pltpu.sync_copy(acc.at[pl.ds(s * zrows, zrows)],
                            p_hbm.at[k].at[pl.ds(s * zrows, zrows)])
            plsc.subcore_barrier()

        for ci in range(per_core):
            @pl.when(c == 0)
            def _():
                process(ci)

            @pl.when(c == 1)
            def _():
                process(per_core + ci)

    return scatter


def _mm0_call(x, deg16, W, R, bn, npad):
    n, d_in = x.shape
    dh = W.shape[1]
    nch = dh // CW
    grid = n // bn

    def body(x_ref, deg_ref, w_ref, r_ref, g_ref, res_ref, dv_ref):
        i = pl.program_id(0)
        cnt = (deg_ref[pl.ds(i * bn, bn), 0]
               + deg_ref[pl.ds(npad + i * bn, bn), 0] + 1.0)
        dv = lax.rsqrt(cnt)[:, None]
        dv_ref[...] = dv
        xb = x_ref[...]
        g = jnp.dot(xb, w_ref[...], preferred_element_type=jnp.float32)
        g = g * dv
        for k in range(nch):
            g_ref[k] = g[:, k * CW:(k + 1) * CW]
        res_ref[...] = jnp.dot(xb, r_ref[...],
                               preferred_element_type=jnp.float32)

    return pl.pallas_call(
        body,
        grid=(grid,),
        in_specs=[
            pl.BlockSpec((bn, d_in), lambda i: (i, 0)),
            pl.BlockSpec((NCORE * npad, 16), lambda i: (0, 0)),
            pl.BlockSpec((d_in, dh), lambda i: (0, 0)),
            pl.BlockSpec((d_in, dh), lambda i: (0, 0)),
        ],
        out_specs=[
            pl.BlockSpec((nch, bn, CW), lambda i: (0, i, 0)),
            pl.BlockSpec((bn, dh), lambda i: (i, 0)),
            pl.BlockSpec((bn, 1), lambda i: (i, 0)),
        ],
        out_shape=[
            jax.ShapeDtypeStruct((nch, n, CW), jnp.float32),
            jax.ShapeDtypeStruct((n, dh), jnp.float32),
            jax.ShapeDtypeStruct((n, 1), jnp.float32),
        ],
    )(x, deg16, W, R)


def _comb_mm_call(p_ch, res, dinv, b, rb, W, R, bn):
    nch_in = p_ch.shape[0]
    n = res.shape[0]
    d_in = nch_in * CW
    dh = W.shape[1]
    nch_out = dh // CW
    grid = n // bn

    def body(p_ref, res_ref, dv_ref, b_ref, rb_ref, w_ref, r_ref,
             go_ref, ro_ref):
        h = jnp.concatenate([p_ref[k] for k in range(nch_in)], axis=1)
        h = h * dv_ref[...] + b_ref[...] + res_ref[...] + rb_ref[...]
        h = jnp.maximum(h, 0.0)
        g2 = jnp.dot(h, w_ref[...], preferred_element_type=jnp.float32)
        g2 = g2 * dv_ref[...]
        for k in range(nch_out):
            go_ref[k] = g2[:, k * CW:(k + 1) * CW]
        ro_ref[...] = jnp.dot(h, r_ref[...],
                              preferred_element_type=jnp.float32)

    return pl.pallas_call(
        body,
        grid=(grid,),
        in_specs=[
            pl.BlockSpec((nch_in, bn, CW), lambda i: (0, i, 0)),
            pl.BlockSpec((bn, d_in), lambda i: (i, 0)),
            pl.BlockSpec((bn, 1), lambda i: (i, 0)),
            pl.BlockSpec((d_in,), lambda i: (0,)),
            pl.BlockSpec((d_in,), lambda i: (0,)),
            pl.BlockSpec((d_in, dh), lambda i: (0, 0)),
            pl.BlockSpec((d_in, dh), lambda i: (0, 0)),
        ],
        out_specs=[
            pl.BlockSpec((nch_out, bn, CW), lambda i: (0, i, 0)),
            pl.BlockSpec((bn, dh), lambda i: (i, 0)),
        ],
        out_shape=[
            jax.ShapeDtypeStruct((nch_out, n, CW), jnp.float32),
            jax.ShapeDtypeStruct((n, dh), jnp.float32),
        ],
    )(p_ch, res, dinv, b, rb, W, R)


def _final_call(p_ch, res, dinv, b, rb, bn):
    nch_in = p_ch.shape[0]
    n = res.shape[0]
    d_in = nch_in * CW
    grid = n // bn

    def body(p_ref, res_ref, dv_ref, b_ref, rb_ref, out_ref):
        h = jnp.concatenate([p_ref[k] for k in range(nch_in)], axis=1)
        out_ref[...] = (h * dv_ref[...] + b_ref[...] + res_ref[...]
                        + rb_ref[...])

    return pl.pallas_call(
        body,
        grid=(grid,),
        in_specs=[
            pl.BlockSpec((nch_in, bn, CW), lambda i: (0, i, 0)),
            pl.BlockSpec((bn, d_in), lambda i: (i, 0)),
            pl.BlockSpec((bn, 1), lambda i: (i, 0)),
            pl.BlockSpec((d_in,), lambda i: (0,)),
            pl.BlockSpec((d_in,), lambda i: (0,)),
        ],
        out_specs=pl.BlockSpec((bn, d_in), lambda i: (i, 0)),
        out_shape=jax.ShapeDtypeStruct((n, d_in), jnp.float32),
    )(p_ch, res, dinv, b, rb)


def kernel(x, edge_index, W0, b0, W1, b1, W2, b2, R0, rb0, R1, rb1, R2, rb2):
    n = x.shape[0]
    e = edge_index.shape[1]
    npad = ((n + 1 + 127) // 128) * 128
    gran = NCORE * NSUB * EB
    epad = ((e + gran - 1) // gran) * gran
    bn = 1000 if n % 1000 == 0 else n // 8

    src = jnp.pad(edge_index[0], (0, epad - e))
    dst = jnp.pad(edge_index[1], (0, epad - e), constant_values=n)
    src2 = src.reshape(-1, EB)
    dst2 = dst.reshape(-1, EB)

    deg16 = _make_deg(n, npad, epad)(dst)

    g0, res0, dinv = _mm0_call(x, deg16, W0, R0, bn, npad)
    zeros = jnp.zeros((npad - n, CW), jnp.float32)
    p0 = _make_scatter(g0.shape[0], n, npad, epad)(g0, src2, dst2, zeros)
    g1, res1 = _comb_mm_call(p0, res0, dinv, b0, rb0, W1, R1, bn)
    p1 = _make_scatter(g1.shape[0], n, npad, epad)(g1, src2, dst2, zeros)
    g2, res2 = _comb_mm_call(p1, res1, dinv, b1, rb1, W2, R2, bn)
    p2 = _make_scatter(g2.shape[0], n, npad, epad)(g2, src2, dst2, zeros)
    out = _final_call(p2, res2, dinv, b2, rb2, bn)
    return out
